# baseline (speedup 1.0000x reference)
import functools

import jax
import jax.numpy as jnp
from jax import lax
from jax.experimental import pallas as pl
from jax.experimental.pallas import tpu as pltpu
from jax.experimental.pallas import tpu_sc as plsc

_B = 16384
_F = 26
_D = 32
_NW = 32
_PER_W = _B * _F // _NW
_CHUNK = 128
_NCH = _PER_W // _CHUNK
_LANES = 16
_PAT_ROWS = 13


def _make_kernel():
    mesh = plsc.VectorSubcoreMesh(core_axis_name="c", subcore_axis_name="s")

    @functools.partial(
        pl.kernel,
        mesh=mesh,
        compiler_params=pltpu.CompilerParams(use_tc_tiling_on_sc=False),
        out_type=jax.ShapeDtypeStruct((_B * _F, _D), jnp.float32),
        scratch_types=[
            pltpu.VMEM((_PAT_ROWS, _CHUNK), jnp.int32),
            pltpu.VMEM((_PER_W,), jnp.int32),
            pltpu.VMEM((_NCH, _CHUNK), jnp.int32),
            pltpu.VMEM((_CHUNK, _D), jnp.float32),
            pltpu.SemaphoreType.DMA,
        ],
    )
    def k(cat_hbm, table_hbm, pat_hbm, out_hbm, pat_v, cat_v, idx_v,
          rows_v, sem):
        wid = lax.axis_index("s") * 2 + lax.axis_index("c")
        base = wid * _PER_W

        pltpu.sync_copy(pat_hbm, pat_v)
        pltpu.sync_copy(cat_hbm.at[pl.ds(base, _PER_W)], cat_v)

        def build(j, carry):
            jm = j % _PAT_ROWS
            for i in range(_CHUNK // _LANES):
                s = i * _LANES
                idx_v[j, pl.ds(s, _LANES)] = (
                    cat_v[pl.ds(j * _CHUNK + s, _LANES)]
                    + pat_v[jm, pl.ds(s, _LANES)])
            return carry

        lax.fori_loop(0, _NCH, build, None)

        def gather(j, carry):
            pltpu.async_copy(table_hbm.at[idx_v.at[j]], rows_v, sem).wait()
            pltpu.sync_copy(rows_v, out_hbm.at[pl.ds(base + j * _CHUNK,
                                                     _CHUNK)])
            return carry

        lax.fori_loop(0, _NCH, gather, None)

    return k


_lookup = _make_kernel()


def kernel(categorical_inputs, embedding_weight, offsets):
    cat_flat = categorical_inputs.reshape(-1).astype(jnp.int32)
    pat = jnp.tile(offsets[:-1].astype(jnp.int32),
                   _PAT_ROWS * _CHUNK // _F).reshape(_PAT_ROWS, _CHUNK)
    out = _lookup(cat_flat, embedding_weight, pat)
    return out.reshape(_B, _F, _D)


# speedup vs baseline: 1.0513x; 1.0513x over previous
import functools

import jax
import jax.numpy as jnp
from jax import lax
from jax.experimental import pallas as pl
from jax.experimental.pallas import tpu as pltpu
from jax.experimental.pallas import tpu_sc as plsc

_B = 16384
_F = 26
_D = 32
_NW = 32
_PER_W = _B * _F // _NW
_BLK = 832
_NB = _PER_W // _BLK
_LANES = 16
_NBUF = 3


def _make_kernel():
    mesh = plsc.VectorSubcoreMesh(core_axis_name="c", subcore_axis_name="s")

    @functools.partial(
        pl.kernel,
        mesh=mesh,
        compiler_params=pltpu.CompilerParams(use_tc_tiling_on_sc=False),
        out_type=jax.ShapeDtypeStruct((_B * _F, _D), jnp.float32),
        scratch_types=[
            pltpu.VMEM((_BLK,), jnp.int32),
            pltpu.VMEM((_NB, _BLK), jnp.int32),
            pltpu.VMEM((_NBUF, _BLK, _D), jnp.float32),
            pltpu.SemaphoreType.DMA,
            pltpu.SemaphoreType.DMA,
            pltpu.SemaphoreType.DMA,
            pltpu.SemaphoreType.DMA,
        ],
    )
    def k(cat_hbm, table_hbm, pat_hbm, out_hbm, pat_v, idx_v, rows_v,
          sem_g0, sem_g1, sem_g2, sem_w):
        sem_g = (sem_g0, sem_g1, sem_g2)
        wid = lax.axis_index("s") * 2 + lax.axis_index("c")
        obase = wid * _PER_W

        pltpu.sync_copy(pat_hbm, pat_v)
        pltpu.sync_copy(cat_hbm.at[pl.ds(wid * _NB, _NB)], idx_v)

        def build(b):
            def body(i, carry):
                s = i * _LANES
                idx_v[b, pl.ds(s, _LANES)] = (idx_v[b, pl.ds(s, _LANES)]
                                              + pat_v[pl.ds(s, _LANES)])
                return carry
            lax.fori_loop(0, _BLK // _LANES, body, None)

        def fire(b):
            return pltpu.async_copy(table_hbm.at[idx_v.at[b]],
                                    rows_v.at[b % _NBUF], sem_g[b % _NBUF])

        gh = [None] * _NB
        wh = [None] * _NB
        for b in range(2):
            build(b)
            gh[b] = fire(b)
        for b in range(_NB):
            if b + 2 < _NB:
                if b >= 1:
                    wh[b - 1].wait()
                build(b + 2)
                gh[b + 2] = fire(b + 2)
            gh[b].wait()
            wh[b] = pltpu.async_copy(
                rows_v.at[b % _NBUF],
                out_hbm.at[pl.ds(obase + b * _BLK, _BLK)], sem_w)
        wh[_NB - 2].wait()
        wh[_NB - 1].wait()

    return k


_lookup = _make_kernel()


def kernel(categorical_inputs, embedding_weight, offsets):
    cat_2d = categorical_inputs.reshape(-1).astype(jnp.int32)
    cat_2d = cat_2d.reshape(_B * _F // _BLK, _BLK)
    pat = jnp.tile(offsets[:-1].astype(jnp.int32), _BLK // _F)
    out = _lookup(cat_2d, embedding_weight, pat)
    return out.reshape(_B, _F, _D)


# speedup vs baseline: 1.0823x; 1.0295x over previous
import functools

import jax
import jax.numpy as jnp
from jax import lax
from jax.experimental import pallas as pl
from jax.experimental.pallas import tpu as pltpu
from jax.experimental.pallas import tpu_sc as plsc

_B = 16384
_F = 26
_D = 32
_NW = 32
_BLK = 1024
_BPF = _B // _BLK
_NB = _B * _F // _BLK // _NW
_LANES = 16
_NBUF = 3


def _make_kernel():
    mesh = plsc.VectorSubcoreMesh(core_axis_name="c", subcore_axis_name="s")

    @functools.partial(
        pl.kernel,
        mesh=mesh,
        compiler_params=pltpu.CompilerParams(use_tc_tiling_on_sc=False),
        out_type=jax.ShapeDtypeStruct((_B * _F, _D), jnp.float32),
        scratch_types=[
            pltpu.VMEM((48,), jnp.int32),
            pltpu.VMEM((_NB, _BLK), jnp.int32),
            pltpu.VMEM((_NBUF, _BLK, _D), jnp.float32),
            pltpu.SemaphoreType.DMA,
            pltpu.SemaphoreType.DMA,
            pltpu.SemaphoreType.DMA,
            pltpu.SemaphoreType.DMA,
        ],
    )
    def k(catT_hbm, table_hbm, offs_hbm, out_hbm, offs_v, idx_v, rows_v,
          sem_g0, sem_g1, sem_g2, sem_w):
        sem_g = (sem_g0, sem_g1, sem_g2)
        wid = lax.axis_index("s") * 2 + lax.axis_index("c")
        g0 = wid * _NB

        pltpu.sync_copy(offs_hbm, offs_v)

        def prep(j):
            g = g0 + j
            f = g // _BPF
            c0 = (g % _BPF) * _BLK
            pltpu.sync_copy(catT_hbm.at[f, pl.ds(c0, _BLK)], idx_v.at[j])
            off = offs_v[pl.ds(f, _LANES)][0]

            def body(i, carry):
                s = i * _LANES
                idx_v[j, pl.ds(s, _LANES)] = idx_v[j, pl.ds(s, _LANES)] + off
                return carry

            lax.fori_loop(0, _BLK // _LANES, body, None)

        def fire(b):
            return pltpu.async_copy(table_hbm.at[idx_v.at[b]],
                                    rows_v.at[b % _NBUF], sem_g[b % _NBUF])

        gh = [None] * _NB
        wh = [None] * _NB
        for b in range(2):
            prep(b)
            gh[b] = fire(b)
        for b in range(_NB):
            if b + 2 < _NB:
                if b >= 1:
                    wh[b - 1].wait()
                prep(b + 2)
                gh[b + 2] = fire(b + 2)
            gh[b].wait()
            wh[b] = pltpu.async_copy(
                rows_v.at[b % _NBUF],
                out_hbm.at[pl.ds((g0 + b) * _BLK, _BLK)], sem_w)
        wh[_NB - 2].wait()
        wh[_NB - 1].wait()

    return k


_lookup = _make_kernel()


def kernel(categorical_inputs, embedding_weight, offsets):
    catT = categorical_inputs.T.astype(jnp.int32)
    offs_pad = jnp.pad(offsets[:-1].astype(jnp.int32), (0, 48 - _F))
    out = _lookup(catT, embedding_weight, offs_pad)
    return out.reshape(_F, _B, _D).transpose(1, 0, 2)
